# Initial kernel scaffold; baseline (speedup 1.0000x reference)
import jax
import jax.numpy as jnp
from jax.experimental import pallas as pl


def kernel(spatial_matrix, spatial_embedding):
    raise NotImplementedError("write your pallas kernel here")


# speedup vs baseline: 5.2875x; 5.2875x over previous
import functools

import jax
import jax.numpy as jnp
from jax import lax
from jax.experimental import pallas as pl
from jax.experimental.pallas import tpu as pltpu
from jax.experimental.pallas import tpu_sc as plsc

MAX_PATH = 512
D = 32
BATCH = 8 * 512 * 512
NC, NS, L = 2, 16, 16
NW = NC * NS
PER_W = BATCH // NW
IDX_ROW = 128
CHUNK = 2048
ROWS = CHUNK // IDX_ROW
NCHUNK = PER_W // CHUNK

_mesh = plsc.VectorSubcoreMesh(core_axis_name="c", subcore_axis_name="s")


@functools.partial(
    pl.kernel,
    out_type=jax.ShapeDtypeStruct((BATCH, D), jnp.float32),
    mesh=_mesh,
    scratch_types=[
        pltpu.VMEM((ROWS, IDX_ROW), jnp.int32),
        pltpu.VMEM((CHUNK, D), jnp.float32),
        pltpu.SemaphoreType.DMA,
    ],
    compiler_params=pltpu.CompilerParams(use_tc_tiling_on_sc=False),
)
def _lookup(idx_hbm, table_hbm, out_hbm, idx_v, rows_v, sem):
    wid = lax.axis_index("s") * NC + lax.axis_index("c")
    row0 = wid * (PER_W // IDX_ROW)
    base = wid * PER_W

    def chunk_body(g, _):
        pltpu.sync_copy(idx_hbm.at[pl.ds(row0 + g * ROWS, ROWS)], idx_v)

        def clamp_row(r, _):
            for k in range(IDX_ROW // L):
                v = idx_v[r, pl.ds(k * L, L)]
                idx_v[r, pl.ds(k * L, L)] = jnp.minimum(
                    jnp.maximum(v, 0), MAX_PATH - 1)
            return 0

        lax.fori_loop(0, ROWS, clamp_row, 0, unroll=True)

        def fire(j, _):
            pltpu.async_copy(
                table_hbm.at[idx_v.at[j]],
                rows_v.at[pl.ds(j * IDX_ROW, IDX_ROW)],
                sem,
            )
            return 0

        lax.fori_loop(0, ROWS, fire, 0)

        def drain(j, _):
            pltpu.make_async_copy(
                table_hbm.at[idx_v.at[j]],
                rows_v.at[pl.ds(j * IDX_ROW, IDX_ROW)],
                sem,
            ).wait()
            return 0

        lax.fori_loop(0, ROWS, drain, 0)

        pltpu.sync_copy(rows_v, out_hbm.at[pl.ds(base + g * CHUNK, CHUNK)])
        return 0

    lax.fori_loop(0, NCHUNK, chunk_body, 0)


def kernel(spatial_matrix, spatial_embedding):
    idx = spatial_matrix.reshape(BATCH // IDX_ROW, IDX_ROW)
    out = _lookup(idx, spatial_embedding)
    return out.reshape(spatial_matrix.shape + (D,))


# speedup vs baseline: 5.3364x; 1.0092x over previous
import functools

import jax
import jax.numpy as jnp
from jax import lax
from jax.experimental import pallas as pl
from jax.experimental.pallas import tpu as pltpu
from jax.experimental.pallas import tpu_sc as plsc

MAX_PATH = 512
D = 32
BATCH = 8 * 512 * 512
NC, NS, L = 2, 16, 16
NW = NC * NS
PER_W = BATCH // NW
IDX_ROW = 128
CHUNK = 1024
ROWS = CHUNK // IDX_ROW
NCHUNK = PER_W // CHUNK

_mesh = plsc.VectorSubcoreMesh(core_axis_name="c", subcore_axis_name="s")


@functools.partial(
    pl.kernel,
    out_type=jax.ShapeDtypeStruct((BATCH, D), jnp.float32),
    mesh=_mesh,
    scratch_types=[
        pltpu.VMEM((ROWS, IDX_ROW), jnp.int32),
        pltpu.VMEM((ROWS, IDX_ROW), jnp.int32),
        pltpu.VMEM((CHUNK, D), jnp.float32),
        pltpu.VMEM((CHUNK, D), jnp.float32),
        pltpu.SemaphoreType.DMA,
        pltpu.SemaphoreType.DMA,
        pltpu.SemaphoreType.DMA,
        pltpu.SemaphoreType.DMA,
    ],
    compiler_params=pltpu.CompilerParams(use_tc_tiling_on_sc=False),
)
def _lookup(idx_hbm, table_hbm, out_hbm,
            idx0, idx1, rows0, rows1, gs0, gs1, ws0, ws1):
    idxb = (idx0, idx1)
    rowsb = (rows0, rows1)
    gs = (gs0, gs1)
    ws = (ws0, ws1)

    wid = lax.axis_index("s") * NC + lax.axis_index("c")
    row0 = wid * (PER_W // IDX_ROW)
    base = wid * PER_W

    def stage(c, b):
        pltpu.sync_copy(idx_hbm.at[pl.ds(row0 + c * ROWS, ROWS)], idxb[b])
        for r in range(ROWS):
            for k in range(IDX_ROW // L):
                v = idxb[b][r, pl.ds(k * L, L)]
                idxb[b][r, pl.ds(k * L, L)] = jnp.minimum(
                    jnp.maximum(v, 0), MAX_PATH - 1)
        for j in range(ROWS):
            pltpu.async_copy(
                table_hbm.at[idxb[b].at[j]],
                rowsb[b].at[pl.ds(j * IDX_ROW, IDX_ROW)],
                gs[b],
            )

    def drain_gather(b):
        for j in range(ROWS):
            pltpu.make_async_copy(
                table_hbm.at[idxb[b].at[j]],
                rowsb[b].at[pl.ds(j * IDX_ROW, IDX_ROW)],
                gs[b],
            ).wait()

    def fire_writeback(c, b):
        pltpu.async_copy(rowsb[b], out_hbm.at[pl.ds(base + c * CHUNK, CHUNK)],
                         ws[b])

    def wait_writeback(c, b):
        pltpu.make_async_copy(
            rowsb[b], out_hbm.at[pl.ds(base + c * CHUNK, CHUNK)],
            ws[b]).wait()

    stage(0, 0)

    def pair_body(g2, _):
        g = g2 * 2
        for b in range(2):
            c = g + b
            nb = 1 - b
            drain_gather(b)
            fire_writeback(c, b)

            @pl.when(c + 1 < NCHUNK)
            def _():
                @pl.when(c >= 1)
                def _():
                    wait_writeback(c - 1, nb)

                stage(c + 1, nb)

        return 0

    lax.fori_loop(0, NCHUNK // 2, pair_body, 0)

    wait_writeback(NCHUNK - 2, 0)
    wait_writeback(NCHUNK - 1, 1)


def kernel(spatial_matrix, spatial_embedding):
    idx = spatial_matrix.reshape(BATCH // IDX_ROW, IDX_ROW)
    out = _lookup(idx, spatial_embedding)
    return out.reshape(spatial_matrix.shape + (D,))


# speedup vs baseline: 6.9731x; 1.3067x over previous
import functools

import jax
import jax.numpy as jnp
from jax import lax
from jax.experimental import pallas as pl
from jax.experimental.pallas import tpu as pltpu
from jax.experimental.pallas import tpu_sc as plsc

MAX_PATH = 512
D = 32
B0, N = 8, 512
NC, NS, L = 2, 16, 16
NW = NC * NS
W_PER_B = NW // B0
ROWS_W = N // W_PER_B
CR = 2
CHUNK = CR * N
ROWS = CHUNK // 128
NCHUNK = ROWS_W // CR

_mesh = plsc.VectorSubcoreMesh(core_axis_name="c", subcore_axis_name="s")


@functools.partial(
    pl.kernel,
    out_type=jax.ShapeDtypeStruct((B0, N, N, D), jnp.float32),
    mesh=_mesh,
    scratch_types=[
        pltpu.VMEM((CR, N), jnp.int32),
        pltpu.VMEM((CR, N), jnp.int32),
        pltpu.VMEM((CR, N, D), jnp.float32),
        pltpu.VMEM((CR, N, D), jnp.float32),
        pltpu.SemaphoreType.DMA,
        pltpu.SemaphoreType.DMA,
        pltpu.SemaphoreType.DMA,
        pltpu.SemaphoreType.DMA,
    ],
    compiler_params=pltpu.CompilerParams(use_tc_tiling_on_sc=False),
)
def _lookup(idx_hbm, table_hbm, out_hbm,
            idx0, idx1, rows0, rows1, gs0, gs1, ws0, ws1):
    idxb = (idx0, idx1)
    rowsb = (rows0, rows1)
    gs = (gs0, gs1)
    ws = (ws0, ws1)

    wid = lax.axis_index("s") * NC + lax.axis_index("c")
    bi = wid // W_PER_B
    r0 = (wid % W_PER_B) * ROWS_W
    toff = wid * MAX_PATH

    def stage(c, b):
        pltpu.sync_copy(idx_hbm.at[bi, pl.ds(r0 + c * CR, CR)], idxb[b])
        for r in range(CR):
            for k in range(N // L):
                v = idxb[b][r, pl.ds(k * L, L)]
                idxb[b][r, pl.ds(k * L, L)] = (
                    jnp.minimum(jnp.maximum(v, 0), MAX_PATH - 1) + toff)
        for j in range(ROWS):
            r, k = divmod(j, N // 128)
            pltpu.async_copy(
                table_hbm.at[idxb[b].at[r, pl.ds(k * 128, 128)]],
                rowsb[b].at[r, pl.ds(k * 128, 128)],
                gs[b],
            )

    def drain_gather(b):
        for j in range(ROWS):
            r, k = divmod(j, N // 128)
            pltpu.make_async_copy(
                table_hbm.at[idxb[b].at[r, pl.ds(k * 128, 128)]],
                rowsb[b].at[r, pl.ds(k * 128, 128)],
                gs[b],
            ).wait()

    def fire_writeback(c, b):
        pltpu.async_copy(rowsb[b], out_hbm.at[bi, pl.ds(r0 + c * CR, CR)],
                         ws[b])

    def wait_writeback(c, b):
        pltpu.make_async_copy(
            rowsb[b], out_hbm.at[bi, pl.ds(r0 + c * CR, CR)], ws[b]).wait()

    stage(0, 0)

    def pair_body(g2, _):
        g = g2 * 2
        for b in range(2):
            c = g + b
            nb = 1 - b
            drain_gather(b)
            fire_writeback(c, b)

            @pl.when(c + 1 < NCHUNK)
            def _():
                @pl.when(c >= 1)
                def _():
                    wait_writeback(c - 1, nb)

                stage(c + 1, nb)

        return 0

    lax.fori_loop(0, NCHUNK // 2, pair_body, 0)

    wait_writeback(NCHUNK - 2, 0)
    wait_writeback(NCHUNK - 1, 1)


def kernel(spatial_matrix, spatial_embedding):
    table_rep = jnp.tile(spatial_embedding, (NW, 1))
    return _lookup(spatial_matrix, table_rep)
